# baseline (speedup 1.0000x reference)
import jax
import jax.numpy as jnp
from jax.experimental import pallas as pl
from jax.experimental.pallas import tpu as pltpu

_B, _C, _H, _W = 1024, 256, 14, 14
_HW = _H * _W
_N = 100000
_BB = 64
_NB = _B // _BB
_NQ = 2
_Q = _HW // _NQ


def _tied_dropout_kernel(idx_ref, xt_ref, *rest):
    mask_refs = rest[:_BB]
    o_ref = rest[_BB]
    bm_ref = rest[_BB + 1]
    b0 = pl.program_id(0) * _BB

    @pl.when(pl.program_id(1) == 0)
    def _():
        for i in range(_BB):
            r = idx_ref[b0 + i]
            grp = mask_refs[i][...].astype(jnp.float32)
            bm_ref[i, :] = pltpu.roll(grp, -(r % 32), axis=0)[0, :]

    o_ref[...] = xt_ref[...] * bm_ref[...][None, :, :]


def _mask_spec(i):
    return pl.BlockSpec(
        (32, _C), lambda b, q, idx_ref, i=i: (idx_ref[b * _BB + i] // 32, 0)
    )


def kernel(X, idx, masks):
    XT = jnp.transpose(X, (2, 3, 0, 1)).reshape(_HW, _B, _C)
    masks = masks.astype(jnp.int8)
    grid_spec = pltpu.PrefetchScalarGridSpec(
        num_scalar_prefetch=1,
        grid=(_NB, _NQ),
        in_specs=[
            pl.BlockSpec((_Q, _BB, _C), lambda b, q, idx_ref: (q, b, 0)),
        ] + [_mask_spec(i) for i in range(_BB)],
        out_specs=pl.BlockSpec((_Q, _BB, _C), lambda b, q, idx_ref: (q, b, 0)),
        scratch_shapes=[pltpu.VMEM((_BB, _C), jnp.float32)],
    )
    out_t = pl.pallas_call(
        _tied_dropout_kernel,
        grid_spec=grid_spec,
        out_shape=jax.ShapeDtypeStruct((_HW, _B, _C), jnp.float32),
    )(idx, XT, *([masks] * _BB))
    return out_t.reshape(_H, _W, _B, _C).transpose(2, 3, 0, 1)


# speedup vs baseline: 1.0176x; 1.0176x over previous
import jax
import jax.numpy as jnp
from jax.experimental import pallas as pl
from jax.experimental.pallas import tpu as pltpu

_B, _C, _H, _W = 1024, 256, 14, 14
_HW = _H * _W
_N = 100000
_BB = 32
_NB = _B // _BB


def _tied_dropout_kernel(idx_ref, xt_ref, *rest):
    mask_refs = rest[:_BB]
    o_ref = rest[_BB]
    bm_ref = rest[_BB + 1]
    b0 = pl.program_id(0) * _BB
    for i in range(_BB):
        r = idx_ref[b0 + i]
        grp = mask_refs[i][...].astype(jnp.float32)
        bm_ref[i, :] = pltpu.roll(grp, -(r % 64), axis=0)[0, :]
    o_ref[...] = xt_ref[...] * bm_ref[...][None, :, :]


def _mask_spec(i):
    return pl.BlockSpec(
        (64, _C), lambda b, idx_ref, i=i: (idx_ref[b * _BB + i] // 64, 0)
    )


def kernel(X, idx, masks):
    XT = jnp.transpose(X, (2, 3, 0, 1)).reshape(_HW, _B, _C)
    masks = masks.astype(jnp.int4)
    grid_spec = pltpu.PrefetchScalarGridSpec(
        num_scalar_prefetch=1,
        grid=(_NB,),
        in_specs=[
            pl.BlockSpec((_HW, _BB, _C), lambda b, idx_ref: (0, b, 0)),
        ] + [_mask_spec(i) for i in range(_BB)],
        out_specs=pl.BlockSpec((_HW, _BB, _C), lambda b, idx_ref: (0, b, 0)),
        scratch_shapes=[pltpu.VMEM((_BB, _C), jnp.float32)],
    )
    out_t = pl.pallas_call(
        _tied_dropout_kernel,
        grid_spec=grid_spec,
        out_shape=jax.ShapeDtypeStruct((_HW, _B, _C), jnp.float32),
    )(idx, XT, *([masks] * _BB))
    return out_t.reshape(_H, _W, _B, _C).transpose(2, 3, 0, 1)


# speedup vs baseline: 1.0178x; 1.0002x over previous
import jax
import jax.numpy as jnp
from jax.experimental import pallas as pl
from jax.experimental.pallas import tpu as pltpu

_B, _C, _H, _W = 1024, 256, 14, 14
_HW = _H * _W
_N = 100000
_BB = 32
_NB = _B // _BB


def _tied_dropout_kernel(idx_ref, xt_ref, tab_ref, o_ref, bm_ref):
    b0 = pl.program_id(0) * _BB
    for i in range(_BB):
        r = idx_ref[b0 + i]
        g = pl.multiple_of((r // 64) * 64, 64)
        grp = tab_ref[pl.ds(g, 64), :].astype(jnp.float32)
        bm_ref[i, :] = pltpu.roll(grp, -(r % 64), axis=0)[0, :]
    o_ref[...] = xt_ref[...] * bm_ref[...][None, :, :]


def kernel(X, idx, masks):
    XT = jnp.transpose(X, (2, 3, 0, 1)).reshape(_HW, _B, _C)
    masks = masks.astype(jnp.int4)
    grid_spec = pltpu.PrefetchScalarGridSpec(
        num_scalar_prefetch=1,
        grid=(_NB,),
        in_specs=[
            pl.BlockSpec((_HW, _BB, _C), lambda b, idx_ref: (0, b, 0)),
            pl.BlockSpec((_N, _C), lambda b, idx_ref: (0, 0)),
        ],
        out_specs=pl.BlockSpec((_HW, _BB, _C), lambda b, idx_ref: (0, b, 0)),
        scratch_shapes=[pltpu.VMEM((_BB, _C), jnp.float32)],
    )
    out_t = pl.pallas_call(
        _tied_dropout_kernel,
        grid_spec=grid_spec,
        out_shape=jax.ShapeDtypeStruct((_HW, _B, _C), jnp.float32),
    )(idx, XT, masks)
    return out_t.reshape(_H, _W, _B, _C).transpose(2, 3, 0, 1)
